# baseline (speedup 1.0000x reference)
import functools

import jax
import jax.numpy as jnp
from jax import lax
from jax.experimental import pallas as pl
from jax.experimental.pallas import tpu as pltpu
from jax.experimental.pallas import tpu_sc as plsc

N = 10000
N_PAD = 10240
E = 320000
IN_F = 128
HID = 128
CLS = 64

NC = 2
NS = 16
NW = NC * NS
K = 128
NCHUNK = 80
E_PER_W = K * NCHUNK
E_PAD = E_PER_W * NW
RPS = N_PAD // NS

BLK = 1024


def _sc_degree(dst):
    mesh = plsc.VectorSubcoreMesh(core_axis_name="c", subcore_axis_name="s")
    e0 = jnp.zeros((K, HID), jnp.float32).at[:, 0].set(1.0)
    zrows = jnp.zeros((RPS, HID), jnp.float32)

    @functools.partial(
        pl.kernel,
        out_type=jax.ShapeDtypeStruct((NC, N_PAD, HID), jnp.float32),
        mesh=mesh,
        scratch_types=[
            pltpu.VMEM((K,), jnp.int32),
            pltpu.VMEM((K,), jnp.int32),
            pltpu.VMEM((K, HID), jnp.float32),
            pltpu.VMEM_SHARED((N_PAD, HID), jnp.float32),
        ],
    )
    def k(dst_hbm, e0_hbm, z_hbm, out_hbm, dst_v0, dst_v1, ones_v, acc):
        c = lax.axis_index("c")
        s = lax.axis_index("s")
        w = c * NS + s
        ebase = w * E_PER_W
        pltpu.sync_copy(z_hbm, acc.at[pl.ds(s * RPS, RPS)])
        pltpu.sync_copy(e0_hbm, ones_v)
        plsc.subcore_barrier()

        @pl.loop(0, NCHUNK, step=2)
        def _(i):
            pltpu.sync_copy(dst_hbm.at[pl.ds(ebase + i * K, K)], dst_v0)
            pltpu.sync_copy(ones_v, acc.at[dst_v0], add=True)
            pltpu.sync_copy(dst_hbm.at[pl.ds(ebase + (i + 1) * K, K)], dst_v1)
            pltpu.sync_copy(ones_v, acc.at[dst_v1], add=True)

        plsc.subcore_barrier()
        pltpu.sync_copy(acc.at[pl.ds(s * RPS, RPS)],
                        out_hbm.at[c, pl.ds(s * RPS, RPS)])

    return k(dst, e0, zrows)


def _sc_scatter(table, src, dst, d):
    mesh = plsc.VectorSubcoreMesh(core_axis_name="c", subcore_axis_name="s")
    zrows = jnp.zeros((RPS, d), jnp.float32)

    @functools.partial(
        pl.kernel,
        out_type=jax.ShapeDtypeStruct((NC, N_PAD, d), jnp.float32),
        mesh=mesh,
        scratch_types=[
            pltpu.VMEM((K,), jnp.int32),
            pltpu.VMEM((K,), jnp.int32),
            pltpu.VMEM((K,), jnp.int32),
            pltpu.VMEM((K,), jnp.int32),
            pltpu.VMEM((K, d), jnp.float32),
            pltpu.VMEM((K, d), jnp.float32),
            pltpu.VMEM_SHARED((N_PAD, d), jnp.float32),
            pltpu.SemaphoreType.DMA,
            pltpu.SemaphoreType.DMA,
        ],
    )
    def k(table_hbm, src_hbm, dst_hbm, z_hbm, out_hbm,
          src_v0, src_v1, dst_v0, dst_v1, rows_v0, rows_v1, acc,
          sem0, sem1):
        c = lax.axis_index("c")
        s = lax.axis_index("s")
        w = c * NS + s
        ebase = w * E_PER_W
        pltpu.sync_copy(z_hbm, acc.at[pl.ds(s * RPS, RPS)])
        plsc.subcore_barrier()

        pltpu.sync_copy(src_hbm.at[pl.ds(ebase, K)], src_v0)
        pltpu.make_async_copy(table_hbm.at[src_v0], rows_v0, sem0).start()

        @pl.loop(0, NCHUNK, step=2)
        def _(i):
            pltpu.sync_copy(src_hbm.at[pl.ds(ebase + (i + 1) * K, K)], src_v1)
            pltpu.make_async_copy(table_hbm.at[src_v1], rows_v1, sem1).start()
            pltpu.sync_copy(dst_hbm.at[pl.ds(ebase + i * K, K)], dst_v0)
            pltpu.make_async_copy(table_hbm.at[src_v0], rows_v0, sem0).wait()
            pltpu.sync_copy(rows_v0, acc.at[dst_v0], add=True)

            @pl.when(i + 2 < NCHUNK)
            def _():
                pltpu.sync_copy(
                    src_hbm.at[pl.ds(ebase + (i + 2) * K, K)], src_v0)
                pltpu.make_async_copy(table_hbm.at[src_v0], rows_v0,
                                      sem0).start()

            pltpu.sync_copy(dst_hbm.at[pl.ds(ebase + (i + 1) * K, K)], dst_v1)
            pltpu.make_async_copy(table_hbm.at[src_v1], rows_v1, sem1).wait()
            pltpu.sync_copy(rows_v1, acc.at[dst_v1], add=True)

        plsc.subcore_barrier()
        pltpu.sync_copy(acc.at[pl.ds(s * RPS, RPS)],
                        out_hbm.at[c, pl.ds(s * RPS, RPS)])

    return k(table, src, dst, zrows)


def _tc_layer1(deg_p, x_pad, w1):
    def body(degp_ref, x_ref, w_ref, g_ref, dinv_ref):
        deg = degp_ref[0, :, 0] + degp_ref[1, :, 0] + 1.0
        dinv = lax.rsqrt(deg)
        h = jnp.dot(x_ref[...], w_ref[...], preferred_element_type=jnp.float32)
        g_ref[...] = h * dinv[:, None]
        dinv_ref[...] = dinv

    return pl.pallas_call(
        body,
        grid=(N_PAD // BLK,),
        in_specs=[
            pl.BlockSpec((NC, BLK, HID), lambda i: (0, i, 0)),
            pl.BlockSpec((BLK, IN_F), lambda i: (i, 0)),
            pl.BlockSpec((IN_F, HID), lambda i: (0, 0)),
        ],
        out_specs=[
            pl.BlockSpec((BLK, HID), lambda i: (i, 0)),
            pl.BlockSpec((BLK,), lambda i: (i,)),
        ],
        out_shape=[
            jax.ShapeDtypeStruct((N_PAD, HID), jnp.float32),
            jax.ShapeDtypeStruct((N_PAD,), jnp.float32),
        ],
    )(deg_p, x_pad, w1)


def _tc_layer2(s1_p, g1, dinv, b1, w2):
    def body(sp_ref, g1_ref, dinv_ref, b1_ref, w_ref, g2_ref):
        dinv = dinv_ref[...]
        z = (sp_ref[0] + sp_ref[1] + g1_ref[...]) * dinv[:, None] + b1_ref[...]
        z = jnp.maximum(z, 0.0)
        h = jnp.dot(z, w_ref[...], preferred_element_type=jnp.float32)
        g2_ref[...] = h * dinv[:, None]

    return pl.pallas_call(
        body,
        grid=(N_PAD // BLK,),
        in_specs=[
            pl.BlockSpec((NC, BLK, HID), lambda i: (0, i, 0)),
            pl.BlockSpec((BLK, HID), lambda i: (i, 0)),
            pl.BlockSpec((BLK,), lambda i: (i,)),
            pl.BlockSpec((HID,), lambda i: (0,)),
            pl.BlockSpec((HID, HID), lambda i: (0, 0)),
        ],
        out_specs=pl.BlockSpec((BLK, HID), lambda i: (i, 0)),
        out_shape=jax.ShapeDtypeStruct((N_PAD, HID), jnp.float32),
    )(s1_p, g1, dinv, b1, w2)


def _tc_out(s2_p, g2, dinv, b2):
    def body(sp_ref, g2_ref, dinv_ref, b2_ref, o_ref):
        o_ref[...] = ((sp_ref[0] + sp_ref[1] + g2_ref[...])
                      * dinv_ref[...][:, None] + b2_ref[...])

    return pl.pallas_call(
        body,
        grid=(N_PAD // BLK,),
        in_specs=[
            pl.BlockSpec((NC, BLK, HID), lambda i: (0, i, 0)),
            pl.BlockSpec((BLK, HID), lambda i: (i, 0)),
            pl.BlockSpec((BLK,), lambda i: (i,)),
            pl.BlockSpec((HID,), lambda i: (0,)),
        ],
        out_specs=pl.BlockSpec((BLK, HID), lambda i: (i, 0)),
        out_shape=jax.ShapeDtypeStruct((N_PAD, HID), jnp.float32),
    )(s2_p, g2, dinv, b2)


def kernel(x, edge_index, W1, b1, W2, b2):
    ei = edge_index.astype(jnp.int32)
    src = jnp.pad(ei[0], (0, E_PAD - E))
    dst = jnp.pad(ei[1], (0, E_PAD - E), constant_values=N_PAD - 1)
    x_pad = jnp.pad(x, ((0, N_PAD - N), (0, 0)))
    w2_pad = jnp.pad(W2, ((0, 0), (0, HID - CLS)))
    b2_pad = jnp.pad(b2, ((0, HID - CLS),))

    deg_p = _sc_degree(dst)
    g1, dinv = _tc_layer1(deg_p, x_pad, W1)
    s1_p = _sc_scatter(g1, src, dst, HID)
    g2 = _tc_layer2(s1_p, g1, dinv, b1, w2_pad)
    s2_p = _sc_scatter(g2, src, dst, HID)
    out = _tc_out(s2_p, g2, dinv, b2_pad)
    return out[:N, :CLS]


# speedup vs baseline: 1.0001x; 1.0001x over previous
import functools

import jax
import jax.numpy as jnp
from jax import lax
from jax.experimental import pallas as pl
from jax.experimental.pallas import tpu as pltpu
from jax.experimental.pallas import tpu_sc as plsc

N = 10000
N_PAD = 10240
E = 320000
IN_F = 128
HID = 128
CLS = 64

NC = 2
NS = 16
NW = NC * NS
K = 128
NCHUNK = 80
E_PER_W = K * NCHUNK
E_PAD = E_PER_W * NW
RPS = N_PAD // NS

BLK = 1024


def _sc_degree(dst):
    mesh = plsc.VectorSubcoreMesh(core_axis_name="c", subcore_axis_name="s")
    e0 = jnp.zeros((K, HID), jnp.float32).at[:, 0].set(1.0)
    zrows = jnp.zeros((RPS, HID), jnp.float32)

    @functools.partial(
        pl.kernel,
        out_type=jax.ShapeDtypeStruct((NC, N_PAD, HID), jnp.float32),
        mesh=mesh,
        scratch_types=[
            pltpu.VMEM((K,), jnp.int32),
            pltpu.VMEM((K,), jnp.int32),
            pltpu.VMEM((K, HID), jnp.float32),
            pltpu.VMEM_SHARED((N_PAD, HID), jnp.float32),
        ],
    )
    def k(dst_hbm, e0_hbm, z_hbm, out_hbm, dst_v0, dst_v1, ones_v, acc):
        c = lax.axis_index("c")
        s = lax.axis_index("s")
        w = c * NS + s
        ebase = w * E_PER_W
        pltpu.sync_copy(z_hbm, acc.at[pl.ds(s * RPS, RPS)])
        pltpu.sync_copy(e0_hbm, ones_v)
        plsc.subcore_barrier()

        @pl.loop(0, NCHUNK, step=2)
        def _(i):
            pltpu.sync_copy(dst_hbm.at[pl.ds(ebase + i * K, K)], dst_v0)
            pltpu.sync_copy(ones_v, acc.at[dst_v0], add=True)
            pltpu.sync_copy(dst_hbm.at[pl.ds(ebase + (i + 1) * K, K)], dst_v1)
            pltpu.sync_copy(ones_v, acc.at[dst_v1], add=True)

        plsc.subcore_barrier()
        pltpu.sync_copy(acc.at[pl.ds(s * RPS, RPS)],
                        out_hbm.at[c, pl.ds(s * RPS, RPS)])

    return k(dst, e0, zrows)


def _sc_scatter(table, src, dst, d):
    mesh = plsc.VectorSubcoreMesh(core_axis_name="c", subcore_axis_name="s")
    zrows = jnp.zeros((RPS, d), jnp.float32)

    @functools.partial(
        pl.kernel,
        out_type=jax.ShapeDtypeStruct((NC, N_PAD, d), jnp.float32),
        mesh=mesh,
        scratch_types=[
            pltpu.VMEM((K,), jnp.int32),
            pltpu.VMEM((K,), jnp.int32),
            pltpu.VMEM((K,), jnp.int32),
            pltpu.VMEM((K,), jnp.int32),
            pltpu.VMEM((K, d), jnp.float32),
            pltpu.VMEM((K, d), jnp.float32),
            pltpu.VMEM_SHARED((N_PAD, d), jnp.float32),
            pltpu.SemaphoreType.DMA,
            pltpu.SemaphoreType.DMA,
        ],
    )
    def k(table_hbm, src_hbm, dst_hbm, z_hbm, out_hbm,
          src_v0, src_v1, dst_v0, dst_v1, rows_v0, rows_v1, acc,
          sem0, sem1):
        c = lax.axis_index("c")
        s = lax.axis_index("s")
        w = c * NS + s
        ebase = w * E_PER_W
        pltpu.sync_copy(z_hbm, acc.at[pl.ds(s * RPS, RPS)])
        plsc.subcore_barrier()

        pltpu.sync_copy(src_hbm.at[pl.ds(ebase, K)], src_v0)
        pltpu.make_async_copy(table_hbm.at[src_v0], rows_v0, sem0).start()

        @pl.loop(0, NCHUNK, step=2)
        def _(i):
            pltpu.sync_copy(src_hbm.at[pl.ds(ebase + (i + 1) * K, K)], src_v1)
            pltpu.make_async_copy(table_hbm.at[src_v1], rows_v1, sem1).start()
            pltpu.sync_copy(dst_hbm.at[pl.ds(ebase + i * K, K)], dst_v0)
            pltpu.make_async_copy(table_hbm.at[src_v0], rows_v0, sem0).wait()
            pltpu.sync_copy(rows_v0, acc.at[dst_v0], add=True)

            @pl.when(i + 2 < NCHUNK)
            def _():
                pltpu.sync_copy(
                    src_hbm.at[pl.ds(ebase + (i + 2) * K, K)], src_v0)
                pltpu.make_async_copy(table_hbm.at[src_v0], rows_v0,
                                      sem0).start()

            pltpu.sync_copy(dst_hbm.at[pl.ds(ebase + (i + 1) * K, K)], dst_v1)
            pltpu.make_async_copy(table_hbm.at[src_v1], rows_v1, sem1).wait()
            pltpu.sync_copy(rows_v1, acc.at[dst_v1], add=True)

        plsc.subcore_barrier()
        pltpu.sync_copy(acc.at[pl.ds(s * RPS, RPS)],
                        out_hbm.at[c, pl.ds(s * RPS, RPS)])

    return k(table, src, dst, zrows)


def _tc_layer1(deg_p, x_pad, w1):
    def body(degp_ref, x_ref, w_ref, g_ref, dinv_ref):
        deg = degp_ref[0, :, 0] + degp_ref[1, :, 0] + 1.0
        dinv = lax.rsqrt(deg)
        h = jnp.dot(x_ref[...], w_ref[...], preferred_element_type=jnp.float32)
        g_ref[...] = h * dinv[:, None]
        dinv_ref[...] = dinv

    return pl.pallas_call(
        body,
        grid=(N_PAD // BLK,),
        in_specs=[
            pl.BlockSpec((NC, BLK, HID), lambda i: (0, i, 0)),
            pl.BlockSpec((BLK, IN_F), lambda i: (i, 0)),
            pl.BlockSpec((IN_F, HID), lambda i: (0, 0)),
        ],
        out_specs=[
            pl.BlockSpec((BLK, HID), lambda i: (i, 0)),
            pl.BlockSpec((BLK,), lambda i: (i,)),
        ],
        out_shape=[
            jax.ShapeDtypeStruct((N_PAD, HID), jnp.float32),
            jax.ShapeDtypeStruct((N_PAD,), jnp.float32),
        ],
    )(deg_p, x_pad, w1)


def _tc_layer2(s1_p, g1, dinv, b1, w2):
    def body(sp_ref, g1_ref, dinv_ref, b1_ref, w_ref, g2_ref):
        dinv = dinv_ref[...]
        z = (sp_ref[0] + sp_ref[1] + g1_ref[...]) * dinv[:, None] + b1_ref[...]
        z = jnp.maximum(z, 0.0)
        h = jnp.dot(z, w_ref[...], preferred_element_type=jnp.float32)
        g2_ref[...] = h * dinv[:, None]

    return pl.pallas_call(
        body,
        grid=(N_PAD // BLK,),
        in_specs=[
            pl.BlockSpec((NC, BLK, HID), lambda i: (0, i, 0)),
            pl.BlockSpec((BLK, HID), lambda i: (i, 0)),
            pl.BlockSpec((BLK,), lambda i: (i,)),
            pl.BlockSpec((HID,), lambda i: (0,)),
            pl.BlockSpec((HID, HID), lambda i: (0, 0)),
        ],
        out_specs=pl.BlockSpec((BLK, HID), lambda i: (i, 0)),
        out_shape=jax.ShapeDtypeStruct((N_PAD, HID), jnp.float32),
    )(s1_p, g1, dinv, b1, w2)


def _tc_out(s2_p, g2, dinv, b2):
    def body(sp_ref, g2_ref, dinv_ref, b2_ref, o_ref):
        o_ref[...] = ((sp_ref[0] + sp_ref[1] + g2_ref[...])
                      * dinv_ref[...][:, None] + b2_ref[...])

    return pl.pallas_call(
        body,
        grid=(N_PAD // BLK,),
        in_specs=[
            pl.BlockSpec((NC, BLK, HID), lambda i: (0, i, 0)),
            pl.BlockSpec((BLK, HID), lambda i: (i, 0)),
            pl.BlockSpec((BLK,), lambda i: (i,)),
            pl.BlockSpec((HID,), lambda i: (0,)),
        ],
        out_specs=pl.BlockSpec((BLK, HID), lambda i: (i, 0)),
        out_shape=jax.ShapeDtypeStruct((N_PAD, HID), jnp.float32),
    )(s2_p, g2, dinv, b2)


def kernel(x, edge_index, W1, b1, W2, b2):
    ei = edge_index.astype(jnp.int32)
    pad_dst = N + (jnp.arange(E_PAD - E, dtype=jnp.int32) % (N_PAD - N))
    src = jnp.pad(ei[0], (0, E_PAD - E))
    dst = jnp.concatenate([ei[1], pad_dst])
    x_pad = jnp.pad(x, ((0, N_PAD - N), (0, 0)))
    w2_pad = jnp.pad(W2, ((0, 0), (0, HID - CLS)))
    b2_pad = jnp.pad(b2, ((0, HID - CLS),))

    deg_p = _sc_degree(dst)
    g1, dinv = _tc_layer1(deg_p, x_pad, W1)
    s1_p = _sc_scatter(g1, src, dst, HID)
    g2 = _tc_layer2(s1_p, g1, dinv, b1, w2_pad)
    s2_p = _sc_scatter(g2, src, dst, HID)
    out = _tc_out(s2_p, g2, dinv, b2_pad)
    return out[:N, :CLS]


# speedup vs baseline: 2.3552x; 2.3550x over previous
import functools

import jax
import jax.numpy as jnp
from jax import lax
from jax.experimental import pallas as pl
from jax.experimental.pallas import tpu as pltpu
from jax.experimental.pallas import tpu_sc as plsc

N = 10000
N_PAD = 10240
E = 320000
IN_F = 128
HID = 128
CLS = 64

NC = 2
NS = 16
NW = NC * NS
K = 128
NCHUNK = 80
E_PER_W = K * NCHUNK
E_PAD = E_PER_W * NW
RPS = N_PAD // NS

BLK = 1024


def _sc_degree(dst):
    mesh = plsc.VectorSubcoreMesh(core_axis_name="c", subcore_axis_name="s")
    e0 = jnp.zeros((K, HID), jnp.float32).at[:, 0].set(1.0)
    zrows = jnp.zeros((RPS, HID), jnp.float32)

    @functools.partial(
        pl.kernel,
        out_type=jax.ShapeDtypeStruct((NC, N_PAD, HID), jnp.float32),
        mesh=mesh,
        scratch_types=[
            pltpu.VMEM((K,), jnp.int32),
            pltpu.VMEM((K,), jnp.int32),
            pltpu.VMEM((K, HID), jnp.float32),
            pltpu.VMEM_SHARED((N_PAD, HID), jnp.float32),
        ],
    )
    def k(dst_hbm, e0_hbm, z_hbm, out_hbm, dst_v0, dst_v1, ones_v, acc):
        c = lax.axis_index("c")
        s = lax.axis_index("s")
        w = c * NS + s
        ebase = w * E_PER_W
        pltpu.sync_copy(z_hbm, acc.at[pl.ds(s * RPS, RPS)])
        pltpu.sync_copy(e0_hbm, ones_v)
        plsc.subcore_barrier()

        @pl.loop(0, NCHUNK, step=2)
        def _(i):
            pltpu.sync_copy(dst_hbm.at[pl.ds(ebase + i * K, K)], dst_v0)
            pltpu.sync_copy(ones_v, acc.at[dst_v0], add=True)
            pltpu.sync_copy(dst_hbm.at[pl.ds(ebase + (i + 1) * K, K)], dst_v1)
            pltpu.sync_copy(ones_v, acc.at[dst_v1], add=True)

        plsc.subcore_barrier()
        pltpu.sync_copy(acc.at[pl.ds(s * RPS, RPS)],
                        out_hbm.at[c, pl.ds(s * RPS, RPS)])

    return k(dst, e0, zrows)


def _sc_scatter(table, src, dst, d):
    mesh = plsc.VectorSubcoreMesh(core_axis_name="c", subcore_axis_name="s")
    zrows = jnp.zeros((RPS, d), jnp.float32)

    @functools.partial(
        pl.kernel,
        out_type=jax.ShapeDtypeStruct((NC, N_PAD, d), jnp.float32),
        mesh=mesh,
        scratch_types=[
            pltpu.VMEM((K,), jnp.int32),
            pltpu.VMEM((K,), jnp.int32),
            pltpu.VMEM((K,), jnp.int32),
            pltpu.VMEM((K,), jnp.int32),
            pltpu.VMEM((K, d), jnp.float32),
            pltpu.VMEM((K, d), jnp.float32),
            pltpu.VMEM_SHARED((N_PAD, d), jnp.float32),
            pltpu.SemaphoreType.DMA,
            pltpu.SemaphoreType.DMA,
        ],
    )
    def k(table_hbm, src_hbm, dst_hbm, z_hbm, out_hbm,
          src_v0, src_v1, dst_v0, dst_v1, rows_v0, rows_v1, acc,
          sem0, sem1):
        c = lax.axis_index("c")
        s = lax.axis_index("s")
        w = c * NS + s
        ebase = w * E_PER_W
        pltpu.sync_copy(z_hbm, acc.at[pl.ds(s * RPS, RPS)])
        plsc.subcore_barrier()

        pltpu.sync_copy(src_hbm.at[pl.ds(ebase, K)], src_v0)
        pltpu.make_async_copy(table_hbm.at[src_v0], rows_v0, sem0).start()

        @pl.loop(0, NCHUNK, step=2)
        def _(i):
            pltpu.sync_copy(src_hbm.at[pl.ds(ebase + (i + 1) * K, K)], src_v1)
            pltpu.make_async_copy(table_hbm.at[src_v1], rows_v1, sem1).start()
            pltpu.sync_copy(dst_hbm.at[pl.ds(ebase + i * K, K)], dst_v0)
            pltpu.make_async_copy(table_hbm.at[src_v0], rows_v0, sem0).wait()
            pltpu.sync_copy(rows_v0, acc.at[dst_v0], add=True)

            @pl.when(i + 2 < NCHUNK)
            def _():
                pltpu.sync_copy(
                    src_hbm.at[pl.ds(ebase + (i + 2) * K, K)], src_v0)
                pltpu.make_async_copy(table_hbm.at[src_v0], rows_v0,
                                      sem0).start()

            pltpu.sync_copy(dst_hbm.at[pl.ds(ebase + (i + 1) * K, K)], dst_v1)
            pltpu.make_async_copy(table_hbm.at[src_v1], rows_v1, sem1).wait()
            pltpu.sync_copy(rows_v1, acc.at[dst_v1], add=True)

        plsc.subcore_barrier()
        pltpu.sync_copy(acc.at[pl.ds(s * RPS, RPS)],
                        out_hbm.at[c, pl.ds(s * RPS, RPS)])

    return k(table, src, dst, zrows)


def _tc_layer1(deg_p, x_pad, w1):
    def body(degp_ref, x_ref, w_ref, g_ref, dinv_ref):
        deg = degp_ref[0, :, 0] + degp_ref[1, :, 0] + 1.0
        dinv = lax.rsqrt(deg)
        h = jnp.dot(x_ref[...], w_ref[...], preferred_element_type=jnp.float32)
        g_ref[...] = h * dinv[:, None]
        dinv_ref[...] = dinv

    return pl.pallas_call(
        body,
        grid=(N_PAD // BLK,),
        in_specs=[
            pl.BlockSpec((NC, BLK, HID), lambda i: (0, i, 0)),
            pl.BlockSpec((BLK, IN_F), lambda i: (i, 0)),
            pl.BlockSpec((IN_F, HID), lambda i: (0, 0)),
        ],
        out_specs=[
            pl.BlockSpec((BLK, HID), lambda i: (i, 0)),
            pl.BlockSpec((BLK,), lambda i: (i,)),
        ],
        out_shape=[
            jax.ShapeDtypeStruct((N_PAD, HID), jnp.float32),
            jax.ShapeDtypeStruct((N_PAD,), jnp.float32),
        ],
    )(deg_p, x_pad, w1)


def _tc_layer2(s1_p, g1, dinv, b1, w2):
    def body(sp_ref, g1_ref, dinv_ref, b1_ref, w_ref, g2_ref):
        dinv = dinv_ref[...]
        z = (sp_ref[0] + sp_ref[1] + g1_ref[...]) * dinv[:, None] + b1_ref[...]
        z = jnp.maximum(z, 0.0)
        h = jnp.dot(z, w_ref[...], preferred_element_type=jnp.float32)
        g2_ref[...] = h * dinv[:, None]

    return pl.pallas_call(
        body,
        grid=(N_PAD // BLK,),
        in_specs=[
            pl.BlockSpec((NC, BLK, HID), lambda i: (0, i, 0)),
            pl.BlockSpec((BLK, HID), lambda i: (i, 0)),
            pl.BlockSpec((BLK,), lambda i: (i,)),
            pl.BlockSpec((HID,), lambda i: (0,)),
            pl.BlockSpec((HID, HID), lambda i: (0, 0)),
        ],
        out_specs=pl.BlockSpec((BLK, HID), lambda i: (i, 0)),
        out_shape=jax.ShapeDtypeStruct((N_PAD, HID), jnp.float32),
    )(s1_p, g1, dinv, b1, w2)


def _tc_out(s2_p, g2, dinv, b2):
    def body(sp_ref, g2_ref, dinv_ref, b2_ref, o_ref):
        o_ref[...] = ((sp_ref[0] + sp_ref[1] + g2_ref[...])
                      * dinv_ref[...][:, None] + b2_ref[...])

    return pl.pallas_call(
        body,
        grid=(N_PAD // BLK,),
        in_specs=[
            pl.BlockSpec((NC, BLK, HID), lambda i: (0, i, 0)),
            pl.BlockSpec((BLK, HID), lambda i: (i, 0)),
            pl.BlockSpec((BLK,), lambda i: (i,)),
            pl.BlockSpec((HID,), lambda i: (0,)),
        ],
        out_specs=pl.BlockSpec((BLK, HID), lambda i: (i, 0)),
        out_shape=jax.ShapeDtypeStruct((N_PAD, HID), jnp.float32),
    )(s2_p, g2, dinv, b2)


def kernel(x, edge_index, W1, b1, W2, b2):
    ei = edge_index.astype(jnp.int32)
    pad_idx = N + (jnp.arange(E_PAD - E, dtype=jnp.int32) % (N_PAD - N))
    src = jnp.concatenate([ei[0], pad_idx])
    dst = jnp.concatenate([ei[1], pad_idx])
    x_pad = jnp.pad(x, ((0, N_PAD - N), (0, 0)))
    w2_pad = jnp.pad(W2, ((0, 0), (0, HID - CLS)))
    b2_pad = jnp.pad(b2, ((0, HID - CLS),))

    deg_p = _sc_degree(dst)
    g1, dinv = _tc_layer1(deg_p, x_pad, W1)
    s1_p = _sc_scatter(g1, src, dst, HID)
    g2 = _tc_layer2(s1_p, g1, dinv, b1, w2_pad)
    s2_p = _sc_scatter(g2, src, dst, HID)
    out = _tc_out(s2_p, g2, dinv, b2_pad)
    return out[:N, :CLS]


# speedup vs baseline: 3.2124x; 1.3639x over previous
import functools

import jax
import jax.numpy as jnp
from jax import lax
from jax.experimental import pallas as pl
from jax.experimental.pallas import tpu as pltpu
from jax.experimental.pallas import tpu_sc as plsc

N = 10000
N_PAD = 10240
E = 320000
IN_F = 128
HID = 128
CLS = 64

NC = 2
NS = 16
NW = NC * NS
K = 128
NBUF = 2
NCHUNK = 80
E_PER_W = K * NCHUNK
E_PAD = E_PER_W * NW
RPS = N_PAD // NS

BLK = 1024


def _sc_degree(dst):
    mesh = plsc.VectorSubcoreMesh(core_axis_name="c", subcore_axis_name="s")
    e0 = jnp.zeros((K, HID), jnp.float32).at[:, 0].set(1.0)
    zrows = jnp.zeros((RPS, HID), jnp.float32)

    @functools.partial(
        pl.kernel,
        out_type=jax.ShapeDtypeStruct((NC, N_PAD, HID), jnp.float32),
        mesh=mesh,
        scratch_types=[
            [pltpu.VMEM((K,), jnp.int32) for _ in range(NBUF)],
            pltpu.VMEM((K, HID), jnp.float32),
            pltpu.VMEM_SHARED((N_PAD, HID), jnp.float32),
            [pltpu.SemaphoreType.DMA for _ in range(NBUF)],
        ],
    )
    def k(dst_hbm, e0_hbm, z_hbm, out_hbm, dst_vs, ones_v, acc, ssems):
        c = lax.axis_index("c")
        s = lax.axis_index("s")
        w = c * NS + s
        ebase = w * E_PER_W
        pltpu.sync_copy(z_hbm, acc.at[pl.ds(s * RPS, RPS)])
        pltpu.sync_copy(e0_hbm, ones_v)
        plsc.subcore_barrier()

        for b in range(NBUF):
            pltpu.sync_copy(dst_hbm.at[pl.ds(ebase + b * K, K)], dst_vs[b])
            pltpu.make_async_copy(ones_v, acc.at[dst_vs[b]], ssems[b]).start(add=True)

        @pl.loop(NBUF, NCHUNK, step=NBUF)
        def _(i):
            for b in range(NBUF):
                pltpu.make_async_copy(ones_v, acc.at[dst_vs[b]], ssems[b]).wait()
                pltpu.sync_copy(dst_hbm.at[pl.ds(ebase + (i + b) * K, K)],
                                dst_vs[b])
                pltpu.make_async_copy(ones_v, acc.at[dst_vs[b]],
                                      ssems[b]).start(add=True)

        for b in range(NBUF):
            pltpu.make_async_copy(ones_v, acc.at[dst_vs[b]], ssems[b]).wait()
        plsc.subcore_barrier()
        pltpu.sync_copy(acc.at[pl.ds(s * RPS, RPS)],
                        out_hbm.at[c, pl.ds(s * RPS, RPS)])

    return k(dst, e0, zrows)


def _sc_scatter(table, src, dst, d):
    mesh = plsc.VectorSubcoreMesh(core_axis_name="c", subcore_axis_name="s")
    zrows = jnp.zeros((RPS, d), jnp.float32)

    @functools.partial(
        pl.kernel,
        out_type=jax.ShapeDtypeStruct((NC, N_PAD, d), jnp.float32),
        mesh=mesh,
        scratch_types=[
            [pltpu.VMEM((K,), jnp.int32) for _ in range(NBUF)],
            [pltpu.VMEM((K,), jnp.int32) for _ in range(NBUF)],
            [pltpu.VMEM((K, d), jnp.float32) for _ in range(NBUF)],
            pltpu.VMEM_SHARED((N_PAD, d), jnp.float32),
            [pltpu.SemaphoreType.DMA for _ in range(NBUF)],
            [pltpu.SemaphoreType.DMA for _ in range(NBUF)],
        ],
    )
    def k(table_hbm, src_hbm, dst_hbm, z_hbm, out_hbm,
          src_vs, dst_vs, rows_vs, acc, gsems, ssems):
        c = lax.axis_index("c")
        s = lax.axis_index("s")
        w = c * NS + s
        ebase = w * E_PER_W
        pltpu.sync_copy(z_hbm, acc.at[pl.ds(s * RPS, RPS)])
        plsc.subcore_barrier()

        def start_gather(b, chunk):
            pltpu.sync_copy(src_hbm.at[pl.ds(ebase + chunk * K, K)],
                            src_vs[b])
            pltpu.make_async_copy(table_hbm.at[src_vs[b]], rows_vs[b],
                                  gsems[b]).start()

        def start_scatter(b, chunk):
            pltpu.sync_copy(dst_hbm.at[pl.ds(ebase + chunk * K, K)],
                            dst_vs[b])
            pltpu.make_async_copy(table_hbm.at[src_vs[b]], rows_vs[b],
                                  gsems[b]).wait()
            pltpu.make_async_copy(rows_vs[b], acc.at[dst_vs[b]], ssems[b]).start(add=True)

        def wait_scatter(b):
            pltpu.make_async_copy(rows_vs[b], acc.at[dst_vs[b]], ssems[b]).wait()

        for b in range(NBUF):
            start_gather(b, b)

        @pl.loop(0, NCHUNK - NBUF, step=NBUF)
        def _(i):
            for b in range(NBUF):
                start_scatter(b, i + b)
            for b in range(NBUF):
                wait_scatter(b)
                start_gather(b, i + b + NBUF)

        for b in range(NBUF):
            start_scatter(b, NCHUNK - NBUF + b)
        for b in range(NBUF):
            wait_scatter(b)

        plsc.subcore_barrier()
        pltpu.sync_copy(acc.at[pl.ds(s * RPS, RPS)],
                        out_hbm.at[c, pl.ds(s * RPS, RPS)])

    return k(table, src, dst, zrows)


def _tc_layer1(deg_p, x_pad, w1):
    def body(degp_ref, x_ref, w_ref, g_ref, dinv_ref):
        deg = degp_ref[0, :, 0] + degp_ref[1, :, 0] + 1.0
        dinv = lax.rsqrt(deg)
        h = jnp.dot(x_ref[...], w_ref[...], preferred_element_type=jnp.float32)
        g_ref[...] = h * dinv[:, None]
        dinv_ref[...] = dinv

    return pl.pallas_call(
        body,
        grid=(N_PAD // BLK,),
        in_specs=[
            pl.BlockSpec((NC, BLK, HID), lambda i: (0, i, 0)),
            pl.BlockSpec((BLK, IN_F), lambda i: (i, 0)),
            pl.BlockSpec((IN_F, HID), lambda i: (0, 0)),
        ],
        out_specs=[
            pl.BlockSpec((BLK, HID), lambda i: (i, 0)),
            pl.BlockSpec((BLK,), lambda i: (i,)),
        ],
        out_shape=[
            jax.ShapeDtypeStruct((N_PAD, HID), jnp.float32),
            jax.ShapeDtypeStruct((N_PAD,), jnp.float32),
        ],
    )(deg_p, x_pad, w1)


def _tc_layer2(s1_p, g1, dinv, b1, w2):
    def body(sp_ref, g1_ref, dinv_ref, b1_ref, w_ref, g2_ref):
        dinv = dinv_ref[...]
        z = (sp_ref[0] + sp_ref[1] + g1_ref[...]) * dinv[:, None] + b1_ref[...]
        z = jnp.maximum(z, 0.0)
        h = jnp.dot(z, w_ref[...], preferred_element_type=jnp.float32)
        g2_ref[...] = h * dinv[:, None]

    return pl.pallas_call(
        body,
        grid=(N_PAD // BLK,),
        in_specs=[
            pl.BlockSpec((NC, BLK, HID), lambda i: (0, i, 0)),
            pl.BlockSpec((BLK, HID), lambda i: (i, 0)),
            pl.BlockSpec((BLK,), lambda i: (i,)),
            pl.BlockSpec((HID,), lambda i: (0,)),
            pl.BlockSpec((HID, HID), lambda i: (0, 0)),
        ],
        out_specs=pl.BlockSpec((BLK, HID), lambda i: (i, 0)),
        out_shape=jax.ShapeDtypeStruct((N_PAD, HID), jnp.float32),
    )(s1_p, g1, dinv, b1, w2)


def _tc_out(s2_p, g2, dinv, b2):
    def body(sp_ref, g2_ref, dinv_ref, b2_ref, o_ref):
        o_ref[...] = ((sp_ref[0] + sp_ref[1] + g2_ref[...])
                      * dinv_ref[...][:, None] + b2_ref[...])

    return pl.pallas_call(
        body,
        grid=(N_PAD // BLK,),
        in_specs=[
            pl.BlockSpec((NC, BLK, HID), lambda i: (0, i, 0)),
            pl.BlockSpec((BLK, HID), lambda i: (i, 0)),
            pl.BlockSpec((BLK,), lambda i: (i,)),
            pl.BlockSpec((HID,), lambda i: (0,)),
        ],
        out_specs=pl.BlockSpec((BLK, HID), lambda i: (i, 0)),
        out_shape=jax.ShapeDtypeStruct((N_PAD, HID), jnp.float32),
    )(s2_p, g2, dinv, b2)


def kernel(x, edge_index, W1, b1, W2, b2):
    ei = edge_index.astype(jnp.int32)
    pad_idx = N + (jnp.arange(E_PAD - E, dtype=jnp.int32) % (N_PAD - N))
    src = jnp.concatenate([ei[0], pad_idx])
    dst = jnp.concatenate([ei[1], pad_idx])
    x_pad = jnp.pad(x, ((0, N_PAD - N), (0, 0)))
    w2_pad = jnp.pad(W2, ((0, 0), (0, HID - CLS)))
    b2_pad = jnp.pad(b2, ((0, HID - CLS),))

    deg_p = _sc_degree(dst)
    g1, dinv = _tc_layer1(deg_p, x_pad, W1)
    s1_p = _sc_scatter(g1, src, dst, HID)
    g2 = _tc_layer2(s1_p, g1, dinv, b1, w2_pad)
    s2_p = _sc_scatter(g2, src, dst, HID)
    out = _tc_out(s2_p, g2, dinv, b2_pad)
    return out[:N, :CLS]
